# baseline (speedup 1.0000x reference)
import functools
import numpy as np
import jax
import jax.numpy as jnp
from jax import lax
from jax.experimental import pallas as pl
from jax.experimental.pallas import tpu as pltpu
from jax.experimental.pallas import tpu_sc as plsc

_NF = 26
_D = 16
_B = 16384
_TOTAL = _B * _NF
_NW = 32
_PER_W = _TOTAL // _NW
_NCHUNK = 8
_CHUNK = _PER_W // _NCHUNK
_VREGS_W = _PER_W // 16

_FIELD_OFFSETS = np.array((0, *np.cumsum([100000] * _NF)[:-1]), dtype=np.int32)
_OFF_TILE = np.array(
    [[_FIELD_OFFSETS[(16 * j + l) % _NF] for l in range(16)] for j in range(_NF)],
    dtype=np.int32,
).reshape(-1)


def _sc_gather_body(x_hbm, off_hbm, table_hbm, out_hbm, idx_v, off_v, rows_v, sem):
    c = lax.axis_index("c")
    s = lax.axis_index("s")
    wid = s * 2 + c
    base = wid * _PER_W
    pltpu.sync_copy(off_hbm, off_v)
    pltpu.sync_copy(x_hbm.at[pl.ds(base, _PER_W)], idx_v)

    def add_off(j, carry):
        r = lax.rem(j, _NF)
        idx_v[pl.ds(j * 16, 16)] = idx_v[pl.ds(j * 16, 16)] + off_v[pl.ds(r * 16, 16)]
        return carry

    lax.fori_loop(0, _VREGS_W, add_off, 0)

    for ci in range(_NCHUNK):
        start = ci * _CHUNK
        pltpu.async_copy(
            table_hbm.at[idx_v.at[pl.ds(start, _CHUNK)]], rows_v, sem
        ).wait()
        pltpu.sync_copy(rows_v, out_hbm.at[pl.ds(base + start, _CHUNK)])


@functools.cache
def _make_sc_gather():
    return pl.kernel(
        _sc_gather_body,
        out_type=jax.ShapeDtypeStruct((_TOTAL, _D), jnp.float32),
        mesh=plsc.VectorSubcoreMesh(core_axis_name="c", subcore_axis_name="s"),
        scratch_types=[
            pltpu.VMEM((_PER_W,), jnp.int32),
            pltpu.VMEM((_NF * 16,), jnp.int32),
            pltpu.VMEM((_CHUNK, _D), jnp.float32),
            pltpu.SemaphoreType.DMA,
        ],
        compiler_params=pltpu.CompilerParams(use_tc_tiling_on_sc=False),
    )


_BLK = 1024
_NB = _B // _BLK


def _mlp_body(emb_ref, W1_ref, b1_ref, g1_ref, be1_ref, W2_ref, b2_ref, g2_ref,
              be2_ref, Wo_ref, bo_ref, out_ref, h1_s, h2_s, s1_s, q1_s):
    i = pl.program_id(0)
    e = emb_ref[...]
    h1 = jnp.dot(e, W1_ref[...], preferred_element_type=jnp.float32) + b1_ref[...]
    h1_s[pl.ds(i * _BLK, _BLK), :] = h1

    @pl.when(i == 0)
    def _():
        s1_s[...] = jnp.zeros_like(s1_s)
        q1_s[...] = jnp.zeros_like(q1_s)

    s1_s[...] += jnp.sum(h1, axis=0, keepdims=True)
    q1_s[...] += jnp.sum(h1 * h1, axis=0, keepdims=True)

    @pl.when(i == _NB - 1)
    def _():
        mu1 = s1_s[...] * (1.0 / _B)
        var1 = q1_s[...] * (1.0 / _B) - mu1 * mu1
        inv1 = lax.rsqrt(var1 + 1e-5) * g1_ref[...]

        def blk2(k, carry):
            s2, q2 = carry
            z = h1_s[pl.ds(k * _BLK, _BLK), :]
            z = jnp.maximum((z - mu1) * inv1 + be1_ref[...], 0.0)
            h2 = jnp.dot(z, W2_ref[...], preferred_element_type=jnp.float32) + b2_ref[...]
            h2_s[pl.ds(k * _BLK, _BLK), :] = h2
            return (s2 + jnp.sum(h2, axis=0, keepdims=True),
                    q2 + jnp.sum(h2 * h2, axis=0, keepdims=True))

        s2, q2 = lax.fori_loop(
            0, _NB, blk2,
            (jnp.zeros((1, 32), jnp.float32), jnp.zeros((1, 32), jnp.float32)),
        )
        mu2 = s2 * (1.0 / _B)
        var2 = q2 * (1.0 / _B) - mu2 * mu2
        inv2 = lax.rsqrt(var2 + 1e-5) * g2_ref[...]

        def blk3(k, carry):
            z = h2_s[pl.ds(k * _BLK, _BLK), :]
            z = jnp.maximum((z - mu2) * inv2 + be2_ref[...], 0.0)
            y = jnp.dot(z, Wo_ref[...], preferred_element_type=jnp.float32) + bo_ref[...]
            out_ref[pl.ds(k * _BLK, _BLK), :] = 1.0 / (1.0 + jnp.exp(-y))
            return carry

        lax.fori_loop(0, _NB, blk3, 0)


def _mlp(emb, W1, b1, g1, be1, W2, b2, g2, be2, Wo, bo):
    full = lambda shape: pl.BlockSpec(shape, lambda i: (0, 0))
    return pl.pallas_call(
        _mlp_body,
        grid=(_NB,),
        in_specs=[
            pl.BlockSpec((_BLK, _NF * _D), lambda i: (i, 0)),
            full(W1.shape), full(b1.shape), full(g1.shape), full(be1.shape),
            full(W2.shape), full(b2.shape), full(g2.shape), full(be2.shape),
            full(Wo.shape), full(bo.shape),
        ],
        out_specs=pl.BlockSpec((_B, 1), lambda i: (0, 0)),
        out_shape=jax.ShapeDtypeStruct((_B, 1), jnp.float32),
        scratch_shapes=[
            pltpu.VMEM((_B, 64), jnp.float32),
            pltpu.VMEM((_B, 32), jnp.float32),
            pltpu.VMEM((1, 64), jnp.float32),
            pltpu.VMEM((1, 64), jnp.float32),
        ],
    )(emb, W1, b1, g1, be1, W2, b2, g2, be2, Wo, bo)


def kernel(x, table, W1, b1, g1, be1, W2, b2, g2, be2, Wo, bo):
    x_flat = x.reshape(-1)
    off_tile = jnp.asarray(_OFF_TILE)
    table_lin = table.reshape(-1).reshape(table.shape)
    emb = _make_sc_gather()(x_flat, off_tile, table_lin)
    emb = emb.reshape(_B, _NF * _D)
    out = _mlp(
        emb, W1,
        b1.reshape(1, -1), g1.reshape(1, -1), be1.reshape(1, -1),
        W2, b2.reshape(1, -1), g2.reshape(1, -1), be2.reshape(1, -1),
        Wo, bo.reshape(1, -1),
    )
    return out.reshape(_B)


# speedup vs baseline: 2.2434x; 2.2434x over previous
import functools
import numpy as np
import jax
import jax.numpy as jnp
from jax import lax
from jax.experimental import pallas as pl
from jax.experimental.pallas import tpu as pltpu
from jax.experimental.pallas import tpu_sc as plsc

_NF = 26
_D = 16
_B = 16384
_TOTAL = _B * _NF
_NW = 32
_PER_W = _TOTAL // _NW
_NCHUNK = 8
_CHUNK = _PER_W // _NCHUNK
_VREGS_W = _PER_W // 16

_FIELD_OFFSETS = np.array((0, *np.cumsum([100000] * _NF)[:-1]), dtype=np.int32)
_OFF_TILE = np.array(
    [[_FIELD_OFFSETS[(16 * j + l) % _NF] for l in range(16)] for j in range(_NF)],
    dtype=np.int32,
).reshape(-1)


def _sc_gather_body(x_hbm, off_hbm, table_hbm, out_hbm, idx_v, off_v, rows_v, sem):
    c = lax.axis_index("c")
    s = lax.axis_index("s")
    wid = s * 2 + c
    base = wid * _PER_W
    pltpu.sync_copy(off_hbm, off_v)
    pltpu.sync_copy(x_hbm.at[pl.ds(base, _PER_W)], idx_v)

    def add_off(j, carry):
        f = lax.rem(j, _NF)
        r = idx_v[pl.ds(j * 16, 16)] + off_v[pl.ds(f * 16, 16)]
        q = ((r & ~8191) + ((r & 1023) << 3)
             + ((r >> 10) & 7))
        idx_v[pl.ds(j * 16, 16)] = q
        return carry

    lax.fori_loop(0, _VREGS_W, add_off, 0)

    for ci in range(_NCHUNK):
        start = ci * _CHUNK
        pltpu.async_copy(
            table_hbm.at[idx_v.at[pl.ds(start, _CHUNK)]], rows_v, sem
        ).wait()
        pltpu.sync_copy(rows_v, out_hbm.at[pl.ds(base + start, _CHUNK)])


@functools.cache
def _make_sc_gather():
    return pl.kernel(
        _sc_gather_body,
        out_type=jax.ShapeDtypeStruct((_TOTAL, _D), jnp.float32),
        mesh=plsc.VectorSubcoreMesh(core_axis_name="c", subcore_axis_name="s"),
        scratch_types=[
            pltpu.VMEM((_PER_W,), jnp.int32),
            pltpu.VMEM((_NF * 16,), jnp.int32),
            pltpu.VMEM((_CHUNK, _D), jnp.float32),
            pltpu.SemaphoreType.DMA,
        ],
        compiler_params=pltpu.CompilerParams(use_tc_tiling_on_sc=False),
    )


_V = 2600000
_TC = 8192
_TNB = (_V + _TC - 1) // _TC
_VG = _TNB * _TC


def _transpose_body(tt_ref, out_ref):
    e = tt_ref[...]
    iota_r = jax.lax.broadcasted_iota(jnp.int32, (_D, 128), 0)
    iota_c = jax.lax.broadcasted_iota(jnp.int32, (_D, 128), 1)
    acc = jnp.zeros((1024, 128), jnp.float32)
    for dr in range(8):
        sel = (iota_c == iota_r + dr * 16).astype(jnp.float32)
        acc += jax.lax.dot_general(
            e[:, dr * 1024:(dr + 1) * 1024], sel,
            dimension_numbers=(((0,), (0,)), ((), ())),
            preferred_element_type=jnp.float32,
        )
    out_ref[...] = acc


def _transpose_table(table_t):
    return pl.pallas_call(
        _transpose_body,
        grid=(_TNB,),
        in_specs=[pl.BlockSpec((_D, _TC), lambda i: (0, i))],
        out_specs=pl.BlockSpec((1024, 128), lambda i: (i, 0)),
        out_shape=jax.ShapeDtypeStruct((_TNB * 1024, 128), jnp.float32),
    )(table_t)


_BLK = 1024
_NB = _B // _BLK


def _mlp_body(emb_ref, W1_ref, b1_ref, g1_ref, be1_ref, W2_ref, b2_ref, g2_ref,
              be2_ref, Wo_ref, bo_ref, out_ref, h1_s, h2_s, s1_s, q1_s):
    i = pl.program_id(0)
    e = emb_ref[...]
    h1 = jnp.dot(e, W1_ref[...], preferred_element_type=jnp.float32) + b1_ref[...]
    h1_s[pl.ds(i * _BLK, _BLK), :] = h1

    @pl.when(i == 0)
    def _():
        s1_s[...] = jnp.zeros_like(s1_s)
        q1_s[...] = jnp.zeros_like(q1_s)

    s1_s[...] += jnp.sum(h1, axis=0, keepdims=True)
    q1_s[...] += jnp.sum(h1 * h1, axis=0, keepdims=True)

    @pl.when(i == _NB - 1)
    def _():
        mu1 = s1_s[...] * (1.0 / _B)
        var1 = q1_s[...] * (1.0 / _B) - mu1 * mu1
        inv1 = lax.rsqrt(var1 + 1e-5) * g1_ref[...]

        def blk2(k, carry):
            s2, q2 = carry
            z = h1_s[pl.ds(k * _BLK, _BLK), :]
            z = jnp.maximum((z - mu1) * inv1 + be1_ref[...], 0.0)
            h2 = jnp.dot(z, W2_ref[...], preferred_element_type=jnp.float32) + b2_ref[...]
            h2_s[pl.ds(k * _BLK, _BLK), :] = h2
            return (s2 + jnp.sum(h2, axis=0, keepdims=True),
                    q2 + jnp.sum(h2 * h2, axis=0, keepdims=True))

        s2, q2 = lax.fori_loop(
            0, _NB, blk2,
            (jnp.zeros((1, 32), jnp.float32), jnp.zeros((1, 32), jnp.float32)),
        )
        mu2 = s2 * (1.0 / _B)
        var2 = q2 * (1.0 / _B) - mu2 * mu2
        inv2 = lax.rsqrt(var2 + 1e-5) * g2_ref[...]

        def blk3(k, carry):
            z = h2_s[pl.ds(k * _BLK, _BLK), :]
            z = jnp.maximum((z - mu2) * inv2 + be2_ref[...], 0.0)
            y = jnp.dot(z, Wo_ref[...], preferred_element_type=jnp.float32) + bo_ref[...]
            out_ref[pl.ds(k * _BLK, _BLK), :] = 1.0 / (1.0 + jnp.exp(-y))
            return carry

        lax.fori_loop(0, _NB, blk3, 0)


def _mlp(emb, W1, b1, g1, be1, W2, b2, g2, be2, Wo, bo):
    full = lambda shape: pl.BlockSpec(shape, lambda i: (0, 0))
    return pl.pallas_call(
        _mlp_body,
        grid=(_NB,),
        in_specs=[
            pl.BlockSpec((_BLK, _NF * _D), lambda i: (i, 0)),
            full(W1.shape), full(b1.shape), full(g1.shape), full(be1.shape),
            full(W2.shape), full(b2.shape), full(g2.shape), full(be2.shape),
            full(Wo.shape), full(bo.shape),
        ],
        out_specs=pl.BlockSpec((_B, 1), lambda i: (0, 0)),
        out_shape=jax.ShapeDtypeStruct((_B, 1), jnp.float32),
        scratch_shapes=[
            pltpu.VMEM((_B, 64), jnp.float32),
            pltpu.VMEM((_B, 32), jnp.float32),
            pltpu.VMEM((1, 64), jnp.float32),
            pltpu.VMEM((1, 64), jnp.float32),
        ],
    )(emb, W1, b1, g1, be1, W2, b2, g2, be2, Wo, bo)


def kernel(x, table, W1, b1, g1, be1, W2, b2, g2, be2, Wo, bo):
    x_flat = x.reshape(-1)
    off_tile = jnp.asarray(_OFF_TILE)
    table_lin = _transpose_table(table.T).reshape(_VG, _D)
    emb = _make_sc_gather()(x_flat, off_tile, table_lin)
    emb = emb.reshape(_B, _NF * _D)
    out = _mlp(
        emb, W1,
        b1.reshape(1, -1), g1.reshape(1, -1), be1.reshape(1, -1),
        W2, b2.reshape(1, -1), g2.reshape(1, -1), be2.reshape(1, -1),
        Wo, bo.reshape(1, -1),
    )
    return out.reshape(_B)


# speedup vs baseline: 2.4921x; 1.1109x over previous
import functools
import numpy as np
import jax
import jax.numpy as jnp
from jax import lax
from jax.experimental import pallas as pl
from jax.experimental.pallas import tpu as pltpu
from jax.experimental.pallas import tpu_sc as plsc

_NF = 26
_D = 16
_B = 16384
_TOTAL = _B * _NF
_NW = 32
_PER_W = _TOTAL // _NW
_NCHUNK = 8
_CHUNK = _PER_W // _NCHUNK
_VREGS_W = _PER_W // 16

_FIELD_OFFSETS = np.array((0, *np.cumsum([100000] * _NF)[:-1]), dtype=np.int32)
_OFF_TILE = np.array(
    [[_FIELD_OFFSETS[(16 * j + l) % _NF] for l in range(16)] for j in range(_NF)],
    dtype=np.int32,
).reshape(-1)


def _sc_gather_body(x_hbm, off_hbm, table_hbm, out_hbm, idx_v, off_v, rows_v, sem):
    c = lax.axis_index("c")
    s = lax.axis_index("s")
    wid = s * 2 + c
    base = wid * _PER_W
    pltpu.sync_copy(off_hbm, off_v)
    pltpu.sync_copy(x_hbm.at[pl.ds(base, _PER_W)], idx_v)

    def add_off(j, carry):
        f = lax.rem(j, _NF)
        r = idx_v[pl.ds(j * 16, 16)] + off_v[pl.ds(f * 16, 16)]
        q = ((r & ~8191) + ((r & 1023) << 3)
             + ((r >> 10) & 7))
        idx_v[pl.ds(j * 16, 16)] = q
        return carry

    lax.fori_loop(0, _VREGS_W, add_off, 0)

    for ci in range(_NCHUNK):
        start = ci * _CHUNK
        pltpu.async_copy(
            table_hbm.at[idx_v.at[pl.ds(start, _CHUNK)]], rows_v, sem
        ).wait()
        pltpu.sync_copy(rows_v, out_hbm.at[pl.ds(base + start, _CHUNK)])


@functools.cache
def _make_sc_gather():
    return pl.kernel(
        _sc_gather_body,
        out_type=jax.ShapeDtypeStruct((_TOTAL, _D), jnp.float32),
        mesh=plsc.VectorSubcoreMesh(core_axis_name="c", subcore_axis_name="s"),
        scratch_types=[
            pltpu.VMEM((_PER_W,), jnp.int32),
            pltpu.VMEM((_NF * 16,), jnp.int32),
            pltpu.VMEM((_CHUNK, _D), jnp.float32),
            pltpu.SemaphoreType.DMA,
        ],
        compiler_params=pltpu.CompilerParams(use_tc_tiling_on_sc=False),
    )


_V = 2600000
_TC = 32768
_TG = _TC // 8192
_TNB = (_V + _TC - 1) // _TC
_VG = _TNB * _TC


def _transpose_body(tt_ref, out_ref):
    e = tt_ref[...]
    iota_r = jax.lax.broadcasted_iota(jnp.int32, (_D, 128), 0)
    iota_c = jax.lax.broadcasted_iota(jnp.int32, (_D, 128), 1)
    for g in range(_TG):
        acc = jnp.zeros((1024, 128), jnp.float32)
        for dr in range(8):
            sel = (iota_c == iota_r + dr * 16).astype(jnp.float32)
            c0 = (g * 8 + dr) * 1024
            acc += jax.lax.dot_general(
                e[:, c0:c0 + 1024], sel,
                dimension_numbers=(((0,), (0,)), ((), ())),
                preferred_element_type=jnp.float32,
            )
        out_ref[pl.ds(g * 1024, 1024), :] = acc


def _transpose_table(table_t):
    return pl.pallas_call(
        _transpose_body,
        grid=(_TNB,),
        in_specs=[pl.BlockSpec((_D, _TC), lambda i: (0, i))],
        out_specs=pl.BlockSpec((_TG * 1024, 128), lambda i: (i, 0)),
        out_shape=jax.ShapeDtypeStruct((_TNB * _TG * 1024, 128), jnp.float32),
    )(table_t)


_BLK = 1024
_NB = _B // _BLK


def _mlp_body(emb_ref, W1_ref, b1_ref, g1_ref, be1_ref, W2_ref, b2_ref, g2_ref,
              be2_ref, Wo_ref, bo_ref, out_ref, h1_s, h2_s, s1_s, q1_s):
    i = pl.program_id(0)
    e = emb_ref[...]
    h1 = jnp.dot(e, W1_ref[...], preferred_element_type=jnp.float32) + b1_ref[...]
    h1_s[pl.ds(i * _BLK, _BLK), :] = h1

    @pl.when(i == 0)
    def _():
        s1_s[...] = jnp.zeros_like(s1_s)
        q1_s[...] = jnp.zeros_like(q1_s)

    s1_s[...] += jnp.sum(h1, axis=0, keepdims=True)
    q1_s[...] += jnp.sum(h1 * h1, axis=0, keepdims=True)

    @pl.when(i == _NB - 1)
    def _():
        mu1 = s1_s[...] * (1.0 / _B)
        var1 = q1_s[...] * (1.0 / _B) - mu1 * mu1
        inv1 = lax.rsqrt(var1 + 1e-5) * g1_ref[...]

        def blk2(k, carry):
            s2, q2 = carry
            z = h1_s[pl.ds(k * _BLK, _BLK), :]
            z = jnp.maximum((z - mu1) * inv1 + be1_ref[...], 0.0)
            h2 = jnp.dot(z, W2_ref[...], preferred_element_type=jnp.float32) + b2_ref[...]
            h2_s[pl.ds(k * _BLK, _BLK), :] = h2
            return (s2 + jnp.sum(h2, axis=0, keepdims=True),
                    q2 + jnp.sum(h2 * h2, axis=0, keepdims=True))

        s2, q2 = lax.fori_loop(
            0, _NB, blk2,
            (jnp.zeros((1, 32), jnp.float32), jnp.zeros((1, 32), jnp.float32)),
        )
        mu2 = s2 * (1.0 / _B)
        var2 = q2 * (1.0 / _B) - mu2 * mu2
        inv2 = lax.rsqrt(var2 + 1e-5) * g2_ref[...]

        def blk3(k, carry):
            z = h2_s[pl.ds(k * _BLK, _BLK), :]
            z = jnp.maximum((z - mu2) * inv2 + be2_ref[...], 0.0)
            y = jnp.dot(z, Wo_ref[...], preferred_element_type=jnp.float32) + bo_ref[...]
            out_ref[pl.ds(k * _BLK, _BLK), :] = 1.0 / (1.0 + jnp.exp(-y))
            return carry

        lax.fori_loop(0, _NB, blk3, 0)


def _mlp(emb, W1, b1, g1, be1, W2, b2, g2, be2, Wo, bo):
    full = lambda shape: pl.BlockSpec(shape, lambda i: (0, 0))
    return pl.pallas_call(
        _mlp_body,
        grid=(_NB,),
        in_specs=[
            pl.BlockSpec((_BLK, _NF * _D), lambda i: (i, 0)),
            full(W1.shape), full(b1.shape), full(g1.shape), full(be1.shape),
            full(W2.shape), full(b2.shape), full(g2.shape), full(be2.shape),
            full(Wo.shape), full(bo.shape),
        ],
        out_specs=pl.BlockSpec((_B, 1), lambda i: (0, 0)),
        out_shape=jax.ShapeDtypeStruct((_B, 1), jnp.float32),
        scratch_shapes=[
            pltpu.VMEM((_B, 64), jnp.float32),
            pltpu.VMEM((_B, 32), jnp.float32),
            pltpu.VMEM((1, 64), jnp.float32),
            pltpu.VMEM((1, 64), jnp.float32),
        ],
    )(emb, W1, b1, g1, be1, W2, b2, g2, be2, Wo, bo)


def kernel(x, table, W1, b1, g1, be1, W2, b2, g2, be2, Wo, bo):
    x_flat = x.reshape(-1)
    off_tile = jnp.asarray(_OFF_TILE)
    table_lin = _transpose_table(table.T).reshape(_VG, _D)
    emb = _make_sc_gather()(x_flat, off_tile, table_lin)
    emb = emb.reshape(_B, _NF * _D)
    out = _mlp(
        emb, W1,
        b1.reshape(1, -1), g1.reshape(1, -1), be1.reshape(1, -1),
        W2, b2.reshape(1, -1), g2.reshape(1, -1), be2.reshape(1, -1),
        Wo, bo.reshape(1, -1),
    )
    return out.reshape(_B)


# speedup vs baseline: 2.5272x; 1.0141x over previous
import functools
import numpy as np
import jax
import jax.numpy as jnp
from jax import lax
from jax.experimental import pallas as pl
from jax.experimental.pallas import tpu as pltpu
from jax.experimental.pallas import tpu_sc as plsc

_NF = 26
_D = 16
_B = 16384
_TOTAL = _B * _NF
_NW = 32
_PER_W = _TOTAL // _NW
_NCHUNK = 8
_CHUNK = _PER_W // _NCHUNK
_VREGS_W = _PER_W // 16

_FIELD_OFFSETS = np.array((0, *np.cumsum([100000] * _NF)[:-1]), dtype=np.int32)
_OFF_TILE = np.array(
    [[_FIELD_OFFSETS[(16 * j + l) % _NF] for l in range(16)] for j in range(_NF)],
    dtype=np.int32,
).reshape(-1)


def _sc_gather_body(x_hbm, off_hbm, table_hbm, out_hbm, idx_v, off_v, rows_v, sem):
    c = lax.axis_index("c")
    s = lax.axis_index("s")
    wid = s * 2 + c
    base = wid * _PER_W
    pltpu.sync_copy(off_hbm, off_v)
    pltpu.sync_copy(x_hbm.at[pl.ds(base, _PER_W)], idx_v)

    def add_off(j, carry):
        f = lax.rem(j, _NF)
        r = idx_v[pl.ds(j * 16, 16)] + off_v[pl.ds(f * 16, 16)]
        q = ((r & ~8191) + ((r & 1023) << 3)
             + ((r >> 10) & 7))
        idx_v[pl.ds(j * 16, 16)] = q
        return carry

    lax.fori_loop(0, _VREGS_W, add_off, 0)

    for ci in range(_NCHUNK):
        start = ci * _CHUNK
        pltpu.async_copy(
            table_hbm.at[idx_v.at[pl.ds(start, _CHUNK)]], rows_v, sem
        ).wait()
        pltpu.sync_copy(rows_v, out_hbm.at[pl.ds(base + start, _CHUNK)])


@functools.cache
def _make_sc_gather():
    return pl.kernel(
        _sc_gather_body,
        out_type=jax.ShapeDtypeStruct((_TOTAL, _D), jnp.float32),
        mesh=plsc.VectorSubcoreMesh(core_axis_name="c", subcore_axis_name="s"),
        scratch_types=[
            pltpu.VMEM((_PER_W,), jnp.int32),
            pltpu.VMEM((_NF * 16,), jnp.int32),
            pltpu.VMEM((_CHUNK, _D), jnp.float32),
            pltpu.SemaphoreType.DMA,
        ],
        compiler_params=pltpu.CompilerParams(use_tc_tiling_on_sc=False),
    )


_V = 2600000
_TC = 65536
_TG = _TC // 8192
_TNB = (_V + _TC - 1) // _TC
_VG = _TNB * _TC


def _transpose_body(tt_ref, out_ref):
    e = tt_ref[...]
    iota_r = jax.lax.broadcasted_iota(jnp.int32, (_D, 128), 0)
    iota_c = jax.lax.broadcasted_iota(jnp.int32, (_D, 128), 1)
    for g in range(_TG):
        acc = jnp.zeros((1024, 128), jnp.float32)
        for dr in range(8):
            sel = (iota_c == iota_r + dr * 16).astype(jnp.float32)
            c0 = (g * 8 + dr) * 1024
            acc += jax.lax.dot_general(
                e[:, c0:c0 + 1024], sel,
                dimension_numbers=(((0,), (0,)), ((), ())),
                preferred_element_type=jnp.float32,
            )
        out_ref[pl.ds(g * 1024, 1024), :] = acc


def _transpose_table(table_t):
    return pl.pallas_call(
        _transpose_body,
        grid=(_TNB,),
        in_specs=[pl.BlockSpec((_D, _TC), lambda i: (0, i))],
        out_specs=pl.BlockSpec((_TG * 1024, 128), lambda i: (i, 0)),
        out_shape=jax.ShapeDtypeStruct((_TNB * _TG * 1024, 128), jnp.float32),
    )(table_t)


_BLK = 1024
_NB = _B // _BLK


def _mlp_body(emb_ref, W1_ref, b1_ref, g1_ref, be1_ref, W2_ref, b2_ref, g2_ref,
              be2_ref, Wo_ref, bo_ref, out_ref, h1_s, h2_s, s1_s, q1_s):
    i = pl.program_id(0)
    e = emb_ref[...]
    h1 = jnp.dot(e, W1_ref[...], preferred_element_type=jnp.float32) + b1_ref[...]
    h1_s[pl.ds(i * _BLK, _BLK), :] = h1

    @pl.when(i == 0)
    def _():
        s1_s[...] = jnp.zeros_like(s1_s)
        q1_s[...] = jnp.zeros_like(q1_s)

    s1_s[...] += jnp.sum(h1, axis=0, keepdims=True)
    q1_s[...] += jnp.sum(h1 * h1, axis=0, keepdims=True)

    @pl.when(i == _NB - 1)
    def _():
        mu1 = s1_s[...] * (1.0 / _B)
        var1 = q1_s[...] * (1.0 / _B) - mu1 * mu1
        inv1 = lax.rsqrt(var1 + 1e-5) * g1_ref[...]

        def blk2(k, carry):
            s2, q2 = carry
            z = h1_s[pl.ds(k * _BLK, _BLK), :]
            z = jnp.maximum((z - mu1) * inv1 + be1_ref[...], 0.0)
            h2 = jnp.dot(z, W2_ref[...], preferred_element_type=jnp.float32) + b2_ref[...]
            h2_s[pl.ds(k * _BLK, _BLK), :] = h2
            return (s2 + jnp.sum(h2, axis=0, keepdims=True),
                    q2 + jnp.sum(h2 * h2, axis=0, keepdims=True))

        s2, q2 = lax.fori_loop(
            0, _NB, blk2,
            (jnp.zeros((1, 32), jnp.float32), jnp.zeros((1, 32), jnp.float32)),
        )
        mu2 = s2 * (1.0 / _B)
        var2 = q2 * (1.0 / _B) - mu2 * mu2
        inv2 = lax.rsqrt(var2 + 1e-5) * g2_ref[...]

        def blk3(k, carry):
            z = h2_s[pl.ds(k * _BLK, _BLK), :]
            z = jnp.maximum((z - mu2) * inv2 + be2_ref[...], 0.0)
            y = jnp.dot(z, Wo_ref[...], preferred_element_type=jnp.float32) + bo_ref[...]
            out_ref[pl.ds(k * _BLK, _BLK), :] = 1.0 / (1.0 + jnp.exp(-y))
            return carry

        lax.fori_loop(0, _NB, blk3, 0)


def _mlp(emb, W1, b1, g1, be1, W2, b2, g2, be2, Wo, bo):
    full = lambda shape: pl.BlockSpec(shape, lambda i: (0, 0))
    return pl.pallas_call(
        _mlp_body,
        grid=(_NB,),
        in_specs=[
            pl.BlockSpec((_BLK, _NF * _D), lambda i: (i, 0)),
            full(W1.shape), full(b1.shape), full(g1.shape), full(be1.shape),
            full(W2.shape), full(b2.shape), full(g2.shape), full(be2.shape),
            full(Wo.shape), full(bo.shape),
        ],
        out_specs=pl.BlockSpec((_B, 1), lambda i: (0, 0)),
        out_shape=jax.ShapeDtypeStruct((_B, 1), jnp.float32),
        scratch_shapes=[
            pltpu.VMEM((_B, 64), jnp.float32),
            pltpu.VMEM((_B, 32), jnp.float32),
            pltpu.VMEM((1, 64), jnp.float32),
            pltpu.VMEM((1, 64), jnp.float32),
        ],
    )(emb, W1, b1, g1, be1, W2, b2, g2, be2, Wo, bo)


def kernel(x, table, W1, b1, g1, be1, W2, b2, g2, be2, Wo, bo):
    x_flat = x.reshape(-1)
    off_tile = jnp.asarray(_OFF_TILE)
    table_lin = _transpose_table(table.T).reshape(_VG, _D)
    emb = _make_sc_gather()(x_flat, off_tile, table_lin)
    emb = emb.reshape(_B, _NF * _D)
    out = _mlp(
        emb, W1,
        b1.reshape(1, -1), g1.reshape(1, -1), be1.reshape(1, -1),
        W2, b2.reshape(1, -1), g2.reshape(1, -1), be2.reshape(1, -1),
        Wo, bo.reshape(1, -1),
    )
    return out.reshape(_B)
